# baseline (speedup 1.0000x reference)
import functools

import jax
import jax.numpy as jnp
from jax import lax
from jax.experimental import pallas as pl
from jax.experimental.pallas import tpu as pltpu
from jax.experimental.pallas import tpu_sc as plsc

N_FACTORS = 64
PADW = 128
BATCH = 16384

_info = plsc.get_sparse_core_info()
NC = _info.num_cores
NS = _info.num_subcores
LANES = _info.num_lanes
NW = NC * NS
BPW = BATCH // NW
GCHUNK = 128
CH = 256
NPASS = BPW // CH


def _body(user_hbm, book_hbm, uemb_hbm, bemb_hbm, out_hbm,
          uidx_v, bidx_v, upair_v, urows_v, brows_v, out_v, sem_u, sem_b):
  wid = lax.axis_index("s") * NC + lax.axis_index("c")
  base = wid * BPW

  pltpu.sync_copy(user_hbm.at[pl.ds(base, BPW)], uidx_v)
  pltpu.sync_copy(book_hbm.at[pl.ds(base, BPW)], bidx_v)

  for i in range(BPW // LANES):
    sl = pl.ds(i * LANES, LANES)
    upair_v[sl] = lax.shift_right_logical(uidx_v[sl], 1)

  lane = lax.iota(jnp.int32, LANES)
  one = jnp.ones((LANES,), jnp.int32)

  for p in range(NPASS):
    copies = []
    for k in range(CH // GCHUNK):
      isl = pl.ds(p * CH + k * GCHUNK, GCHUNK)
      dsl = pl.ds(k * GCHUNK, GCHUNK)
      copies.append(pltpu.async_copy(
          uemb_hbm.at[upair_v.at[isl]], urows_v.at[dsl], sem_u))
      copies.append(pltpu.async_copy(
          bemb_hbm.at[bidx_v.at[isl]], brows_v.at[dsl], sem_b))
    for c in copies:
      c.wait()

    def group(g, carry):
      rows = g * LANES + lane
      ucol = (uidx_v[pl.ds(p * CH + g * LANES, LANES)] & 1) * N_FACTORS
      col = jnp.zeros((LANES,), jnp.int32)
      acc = jnp.zeros((LANES,), jnp.float32)
      for j in range(N_FACTORS):
        u = plsc.load_gather(urows_v, [rows, ucol])
        b = plsc.load_gather(brows_v, [rows, col])
        acc = acc + u * b
        if j + 1 < N_FACTORS:
          ucol = ucol + one
          col = col + one
      out_v[pl.ds(g * LANES, LANES)] = acc
      return carry

    lax.fori_loop(0, CH // LANES, group, 0)
    pltpu.sync_copy(out_v, out_hbm.at[pl.ds(base + p * CH, CH)])


@jax.jit
def kernel(user, book, user_emb, book_emb):
  pad = ((0, 0), (0, PADW - N_FACTORS))
  up = user_emb.reshape(user_emb.shape[0] // 2, PADW)
  bp = jnp.pad(book_emb, pad)
  mesh = plsc.VectorSubcoreMesh(core_axis_name="c", subcore_axis_name="s")
  run = functools.partial(
      pl.kernel,
      out_type=jax.ShapeDtypeStruct((BATCH,), jnp.float32),
      mesh=mesh,
      compiler_params=pltpu.CompilerParams(needs_layout_passes=False),
      scratch_types=[
          pltpu.VMEM((BPW,), jnp.int32),
          pltpu.VMEM((BPW,), jnp.int32),
          pltpu.VMEM((BPW,), jnp.int32),
          pltpu.VMEM((CH, PADW), jnp.float32),
          pltpu.VMEM((CH, PADW), jnp.float32),
          pltpu.VMEM((CH,), jnp.float32),
          pltpu.SemaphoreType.DMA,
          pltpu.SemaphoreType.DMA,
      ],
  )(_body)
  return run(user.astype(jnp.int32), book.astype(jnp.int32), up, bp)


# speedup vs baseline: 1.0093x; 1.0093x over previous
import functools

import jax
import jax.numpy as jnp
from jax import lax
from jax.experimental import pallas as pl
from jax.experimental.pallas import tpu as pltpu
from jax.experimental.pallas import tpu_sc as plsc

N_FACTORS = 64
PAIR = 2 * N_FACTORS
BATCH = 16384

_info = plsc.get_sparse_core_info()
NC = _info.num_cores
NS = _info.num_subcores
LANES = _info.num_lanes
NW = NC * NS
BPW = BATCH // NW
GCHUNK = 128
CH = 256
NPASS = BPW // CH


def _body(user_hbm, book_hbm, uemb_hbm, bemb_hbm, out_hbm,
          uidx_v, bidx_v, upair_v, bpair_v, urows_v, brows_v, out_v,
          sem_u, sem_b):
  wid = lax.axis_index("s") * NC + lax.axis_index("c")
  base = wid * BPW

  pltpu.sync_copy(user_hbm.at[pl.ds(base, BPW)], uidx_v)
  pltpu.sync_copy(book_hbm.at[pl.ds(base, BPW)], bidx_v)

  for i in range(BPW // LANES):
    sl = pl.ds(i * LANES, LANES)
    upair_v[sl] = lax.shift_right_logical(uidx_v[sl], 1)
    bpair_v[sl] = lax.shift_right_logical(bidx_v[sl], 1)

  lane = lax.iota(jnp.int32, LANES)
  one = jnp.ones((LANES,), jnp.int32)

  for p in range(NPASS):
    copies = []
    for k in range(CH // GCHUNK):
      isl = pl.ds(p * CH + k * GCHUNK, GCHUNK)
      dsl = pl.ds(k * GCHUNK, GCHUNK)
      copies.append(pltpu.async_copy(
          uemb_hbm.at[upair_v.at[isl]], urows_v.at[dsl], sem_u))
      copies.append(pltpu.async_copy(
          bemb_hbm.at[bpair_v.at[isl]], brows_v.at[dsl], sem_b))
    for c in copies:
      c.wait()

    def group(g, carry):
      rows = g * LANES + lane
      ucol = (uidx_v[pl.ds(p * CH + g * LANES, LANES)] & 1) * N_FACTORS
      bcol = (bidx_v[pl.ds(p * CH + g * LANES, LANES)] & 1) * N_FACTORS
      acc = jnp.zeros((LANES,), jnp.float32)
      for j in range(N_FACTORS):
        u = plsc.load_gather(urows_v, [rows, ucol])
        b = plsc.load_gather(brows_v, [rows, bcol])
        acc = acc + u * b
        if j + 1 < N_FACTORS:
          ucol = ucol + one
          bcol = bcol + one
      out_v[pl.ds(g * LANES, LANES)] = acc
      return carry

    lax.fori_loop(0, CH // LANES, group, 0)
    pltpu.sync_copy(out_v, out_hbm.at[pl.ds(base + p * CH, CH)])


@jax.jit
def kernel(user, book, user_emb, book_emb):
  n_users, n_factors = user_emb.shape
  n_books = book_emb.shape[0]
  upr = user_emb.reshape(n_users // 2, PAIR)
  bpr = book_emb.reshape(n_books // 2, PAIR)
  mesh = plsc.VectorSubcoreMesh(core_axis_name="c", subcore_axis_name="s")
  run = functools.partial(
      pl.kernel,
      out_type=jax.ShapeDtypeStruct((BATCH,), jnp.float32),
      mesh=mesh,
      compiler_params=pltpu.CompilerParams(needs_layout_passes=False),
      scratch_types=[
          pltpu.VMEM((BPW,), jnp.int32),
          pltpu.VMEM((BPW,), jnp.int32),
          pltpu.VMEM((BPW,), jnp.int32),
          pltpu.VMEM((BPW,), jnp.int32),
          pltpu.VMEM((CH, PAIR), jnp.float32),
          pltpu.VMEM((CH, PAIR), jnp.float32),
          pltpu.VMEM((CH,), jnp.float32),
          pltpu.SemaphoreType.DMA,
          pltpu.SemaphoreType.DMA,
      ],
  )(_body)
  return run(user.astype(jnp.int32), book.astype(jnp.int32), upr, bpr)
